# baseline (speedup 1.0000x reference)
import functools

import jax
import jax.numpy as jnp
from jax import lax
from jax.experimental import pallas as pl
from jax.experimental.pallas import tpu as pltpu
from jax.experimental.pallas import tpu_sc as plsc

N = 50000
E = 800000
HID = 128

NC = 2
NS = 16
LANE = 16


def _build_seg_sum(n_table_rows, W, NP, P, B):
  GMAX = B // 128 + 1
  CONC = 4
  EPT = E // NS
  NB = EPT // B
  NVR = B // LANE
  ACC_R = NP + 128
  ZCH = ACC_R // 128
  ZPT = -(-ZCH // NS)
  RPT = NP // NS

  mesh = plsc.VectorSubcoreMesh(core_axis_name="c", subcore_axis_name="s")

  @functools.partial(
      pl.kernel,
      out_type=jax.ShapeDtypeStruct((NC * P * NP, W), jnp.float32),
      mesh=mesh,
      scratch_types=[
          pltpu.VMEM((B,), jnp.int32),
          pltpu.VMEM((B,), jnp.int32),
          pltpu.VMEM((GMAX, 128), jnp.int32),
          pltpu.VMEM((GMAX, 128), jnp.int32),
          pltpu.VMEM((CONC * 128, W), jnp.float32),
          pltpu.VMEM_SHARED((ACC_R, W), jnp.float32),
          pltpu.SemaphoreType.DMA,
      ],
      compiler_params=pltpu.CompilerParams(
          needs_layout_passes=False, use_tc_tiling_on_sc=False),
  )
  def kern(src_hbm, dst_hbm, table_hbm, zeros_hbm, out_hbm,
           eb_src, eb_dst, stg_src, stg_dst, rowbuf, accum, sem):
    c = lax.axis_index("c")
    s = lax.axis_index("s")
    pltpu.sync_copy(zeros_hbm, rowbuf.at[pl.ds(0, 128)])
    tile_e0 = s * EPT
    c127 = jnp.full((LANE,), 127, jnp.int32)
    iot = lax.iota(jnp.int32, LANE)
    trash = jnp.full((LANE,), NP, jnp.int32)
    zero16 = jnp.zeros((LANE,), jnp.int32)

    for p in range(P):
      r = c * P + p
      lo = r * NP
      lo_v = jnp.full((LANE,), 1, jnp.int32) * lo
      hi_v = lo_v + NP

      for j in range(ZPT):
        ch = s * ZPT + j

        @pl.when(ch < ZCH)
        def _():
          pltpu.sync_copy(rowbuf.at[pl.ds(0, 128)],
                          accum.at[pl.ds(ch * 128, 128)])

      plsc.subcore_barrier()

      def block_body(blk, carry):
        base = tile_e0 + blk * B
        pltpu.sync_copy(src_hbm.at[pl.ds(base, B)], eb_src)
        pltpu.sync_copy(dst_hbm.at[pl.ds(base, B)], eb_dst)

        def scan_body(i, off):
          d = eb_dst[pl.ds(i * LANE, LANE)]
          sv = eb_src[pl.ds(i * LANE, LANE)]
          m = (d >= lo_v) & (d < hi_v)
          inc = jnp.where(m, 1, 0).astype(jnp.int32)
          pos = off + plsc.cumsum(inc) - 1
          row = lax.shift_right_logical(pos, 7)
          col = lax.bitwise_and(pos, c127)
          plsc.store_scatter(stg_src, [row, col], sv, mask=m)
          plsc.store_scatter(stg_dst, [row, col], d - lo_v, mask=m)
          return off + plsc.all_reduce_population_count(m)

        offv = lax.fori_loop(0, NVR, scan_body,
                             jnp.zeros((LANE,), jnp.int32))
        k = jnp.max(offv)
        ng = lax.shift_right_logical(k + 127, 7)
        kpad_v = jnp.zeros((LANE,), jnp.int32) + ng * 128
        for j in range(8):
          pos = offv + (j * LANE) + iot
          pm = pos < kpad_v
          prow = lax.shift_right_logical(pos, 7)
          pcol = lax.bitwise_and(pos, c127)
          plsc.store_scatter(stg_src, [prow, pcol], zero16, mask=pm)
          plsc.store_scatter(stg_dst, [prow, pcol], trash, mask=pm)

        def flush_chunk(ch, cc):
          base_g = ch * CONC
          rem = ng - base_g
          for j in range(CONC):
            @pl.when(j < rem)
            def _():
              pltpu.async_copy(table_hbm.at[stg_src.at[base_g + j]],
                               rowbuf.at[pl.ds(j * 128, 128)], sem)
          for j in range(CONC):
            @pl.when(j < rem)
            def _():
              pltpu.make_async_copy(table_hbm.at[stg_src.at[base_g + j]],
                                    rowbuf.at[pl.ds(j * 128, 128)], sem).wait()
              pltpu.sync_copy(rowbuf.at[pl.ds(j * 128, 128)],
                              accum.at[stg_dst.at[base_g + j]], add=True)
          return cc

        lax.fori_loop(0, lax.div(ng + (CONC - 1), CONC), flush_chunk, 0)
        return carry

      lax.fori_loop(0, NB, block_body, 0)
      plsc.subcore_barrier()
      pltpu.sync_copy(accum.at[pl.ds(s * RPT, RPT)],
                      out_hbm.at[pl.ds(lo + s * RPT, RPT)])
      plsc.subcore_barrier()

  return kern


def _tc_layer1(s1cnt, x, wcat, b1):
  R = 1000
  grid = (N // R,)

  def body(s1_ref, x_ref, w_ref, b_ref, out_ref):
    s1 = s1_ref[...]
    cnt = jnp.maximum(s1[:, 4:5], 1.0)
    feat = jnp.concatenate([s1[:, 0:4] / cnt, x_ref[...]], axis=1)
    h = lax.dot_general(feat, w_ref[...], (((1,), (1,)), ((), ())),
                        preferred_element_type=jnp.float32)
    out_ref[...] = jnp.maximum(h + b_ref[...], 0.0)

  return pl.pallas_call(
      body,
      grid=grid,
      in_specs=[
          pl.BlockSpec((R, 8), lambda i: (i, 0)),
          pl.BlockSpec((R, 4), lambda i: (i, 0)),
          pl.BlockSpec((HID, 8), lambda i: (0, 0)),
          pl.BlockSpec((1, HID), lambda i: (0, 0)),
      ],
      out_specs=pl.BlockSpec((R, HID), lambda i: (i, 0)),
      out_shape=jax.ShapeDtypeStruct((N, HID), jnp.float32),
  )(s1cnt, x, wcat, b1)


def _tc_layer2_head(s2, h1, cnt, W2l, b2, W2r, Wp, bp, Wc1, bc1, Wc2, bc2):
  R = 1000
  nblk = N // R

  def body(s2_ref, h1_ref, cnt_ref, w2l_ref, w2r_ref, b2_ref,
           wp_ref, bp_ref, wc1_ref, bc1_ref, wc2_ref, bc2_ref,
           out_ref, acc_ref):
    i = pl.program_id(0)

    @pl.when(i == 0)
    def _():
      acc_ref[...] = jnp.zeros_like(acc_ref)

    cnt = jnp.maximum(cnt_ref[...], 1.0)
    mean = s2_ref[...] / cnt
    h = (lax.dot_general(mean, w2l_ref[...], (((1,), (1,)), ((), ())),
                         preferred_element_type=jnp.float32)
         + lax.dot_general(h1_ref[...], w2r_ref[...],
                           (((1,), (1,)), ((), ())),
                           preferred_element_type=jnp.float32)
         + b2_ref[...])
    h2 = jnp.maximum(h, 0.0)
    acc_ref[...] += jnp.sum(h2, axis=0, keepdims=True)

    @pl.when(i == nblk - 1)
    def _():
      pooled = acc_ref[...] / float(N)
      emb = jnp.maximum(
          lax.dot_general(pooled, wp_ref[...], (((1,), (1,)), ((), ())),
                          preferred_element_type=jnp.float32) + bp_ref[...],
          0.0)
      hc = jnp.maximum(
          lax.dot_general(emb, wc1_ref[...], (((1,), (1,)), ((), ())),
                          preferred_element_type=jnp.float32) + bc1_ref[...],
          0.0)
      logit = jnp.sum(hc * wc2_ref[...], axis=1, keepdims=True) + bc2_ref[...]
      out_ref[...] = jax.nn.sigmoid(logit)

  return pl.pallas_call(
      body,
      grid=(nblk,),
      in_specs=[
          pl.BlockSpec((R, HID), lambda i: (i, 0)),
          pl.BlockSpec((R, HID), lambda i: (i, 0)),
          pl.BlockSpec((R, 1), lambda i: (i, 0)),
          pl.BlockSpec((HID, HID), lambda i: (0, 0)),
          pl.BlockSpec((HID, HID), lambda i: (0, 0)),
          pl.BlockSpec((1, HID), lambda i: (0, 0)),
          pl.BlockSpec((256, HID), lambda i: (0, 0)),
          pl.BlockSpec((1, 256), lambda i: (0, 0)),
          pl.BlockSpec((HID, 256), lambda i: (0, 0)),
          pl.BlockSpec((1, HID), lambda i: (0, 0)),
          pl.BlockSpec((1, HID), lambda i: (0, 0)),
          pl.BlockSpec((1, 1), lambda i: (0, 0)),
      ],
      out_specs=pl.BlockSpec((1, 1), lambda i: (0, 0)),
      out_shape=jax.ShapeDtypeStruct((1, 1), jnp.float32),
      scratch_shapes=[pltpu.VMEM((1, HID), jnp.float32)],
  )(s2, h1, cnt, W2l, W2r, b2, Wp, bp, Wc1, bc1, Wc2, bc2)


def kernel(x, edge_index, W1l, b1, W1r, W2l, b2, W2r, Wp, bp, Wc1, bc1,
           Wc2, bc2):
  src = edge_index[0].astype(jnp.int32)
  dst = edge_index[1].astype(jnp.int32)
  x = x.astype(jnp.float32)

  xp = jnp.concatenate(
      [x, jnp.ones((N, 1), jnp.float32), jnp.zeros((N, 3), jnp.float32)],
      axis=1)

  NP1 = 25088
  NP2 = 6912
  B = 2000

  zeros8 = jnp.zeros((128, 8), jnp.float32)
  zeros128 = jnp.zeros((128, HID), jnp.float32)

  seg1 = _build_seg_sum(N, 8, NP1, 1, B)
  s1cnt = seg1(src, dst, xp, zeros8)[:N]

  wcat = jnp.concatenate([W1l, W1r], axis=1)
  h1 = _tc_layer1(s1cnt, x, wcat, b1.reshape(1, HID))

  seg2 = _build_seg_sum(N, HID, NP2, 4, B)
  s2 = seg2(src, dst, h1, zeros128)[:N]

  cnt = s1cnt[:, 4:5]
  prob = _tc_layer2_head(
      s2, h1, cnt, W2l, b2.reshape(1, HID), W2r, Wp, bp.reshape(1, 256),
      Wc1, bc1.reshape(1, HID), Wc2, bc2.reshape(1, 1))
  return prob


# speedup vs baseline: 9.6757x; 9.6757x over previous
import functools

import jax
import jax.numpy as jnp
from jax import lax
from jax.experimental import pallas as pl
from jax.experimental.pallas import tpu as pltpu
from jax.experimental.pallas import tpu_sc as plsc

N = 50000
E = 800000
HID = 128

NC = 2
NS = 16
LANE = 16


def _build_seg_sum(n_table_rows, W, NP, P, B):
  GMAX = 32
  EPT = E // NS
  NB = EPT // B
  NVR = B // LANE
  ACC_R = NP + 128
  ZCH = ACC_R // 128
  ZPT = -(-ZCH // NS)
  RPT = NP // NS
  assert B + 128 <= GMAX * 128

  mesh = plsc.VectorSubcoreMesh(core_axis_name="c", subcore_axis_name="s")

  @functools.partial(
      pl.kernel,
      out_type=jax.ShapeDtypeStruct((NC * P * NP, W), jnp.float32),
      mesh=mesh,
      scratch_types=[
          pltpu.VMEM((B,), jnp.int32),
          pltpu.VMEM((B,), jnp.int32),
          pltpu.VMEM((GMAX, 128), jnp.int32),
          pltpu.VMEM((GMAX, 128), jnp.int32),
          pltpu.VMEM((128, W), jnp.float32),
          pltpu.VMEM_SHARED((ACC_R, W), jnp.float32),
          pltpu.SemaphoreType.DMA,
      ],
      compiler_params=pltpu.CompilerParams(
          needs_layout_passes=False, use_tc_tiling_on_sc=False),
  )
  def kern(src_hbm, dst_hbm, table_hbm, zeros_hbm, out_hbm,
           eb_src, eb_dst, stg_src, stg_dst, rowbuf, accum, sem):
    c = lax.axis_index("c")
    s = lax.axis_index("s")
    tile_e0 = s * EPT
    c127 = jnp.full((LANE,), 127, jnp.int32)
    iot = lax.iota(jnp.int32, LANE)
    trash = jnp.full((LANE,), NP, jnp.int32)
    zero16 = jnp.zeros((LANE,), jnp.int32)

    def flush_range(g0, g1):
      def flush(g, cc):
        rr = lax.bitwise_and(g, GMAX - 1)
        pltpu.async_copy(table_hbm.at[stg_src.at[rr]], rowbuf, sem).wait()
        pltpu.sync_copy(rowbuf, accum.at[stg_dst.at[rr]], add=True)
        return cc
      lax.fori_loop(g0, g1, flush, 0)

    for p in range(P):
      r = c * P + p
      lo = r * NP
      lo_v = jnp.full((LANE,), 1, jnp.int32) * lo
      hi_v = lo_v + NP

      pltpu.sync_copy(zeros_hbm, rowbuf)
      for j in range(ZPT):
        ch = s * ZPT + j

        @pl.when(ch < ZCH)
        def _():
          pltpu.sync_copy(rowbuf, accum.at[pl.ds(ch * 128, 128)])

      plsc.subcore_barrier()

      def block_body(blk, carry):
        offv, fg = carry
        base = tile_e0 + blk * B
        pltpu.sync_copy(src_hbm.at[pl.ds(base, B)], eb_src)
        pltpu.sync_copy(dst_hbm.at[pl.ds(base, B)], eb_dst)

        def scan_body(i, off):
          d = eb_dst[pl.ds(i * LANE, LANE)]
          sv = eb_src[pl.ds(i * LANE, LANE)]
          m = (d >= lo_v) & (d < hi_v)
          inc = jnp.where(m, 1, 0).astype(jnp.int32)
          pos = off + plsc.cumsum(inc) - 1
          row = lax.bitwise_and(lax.shift_right_logical(pos, 7), GMAX - 1)
          col = lax.bitwise_and(pos, c127)
          plsc.store_scatter(stg_src, [row, col], sv, mask=m)
          plsc.store_scatter(stg_dst, [row, col], d - lo_v, mask=m)
          return off + plsc.all_reduce_population_count(m)

        offv2 = lax.fori_loop(0, NVR, scan_body, offv)
        tg = lax.shift_right_logical(jnp.max(offv2), 7)
        flush_range(fg, tg)
        return offv2, tg

      offv, fg = lax.fori_loop(
          0, NB, block_body,
          (jnp.zeros((LANE,), jnp.int32), jnp.int32(0)))

      k = jnp.max(offv)
      ngt = lax.shift_right_logical(k + 127, 7)
      kpad_v = jnp.zeros((LANE,), jnp.int32) + ngt * 128
      for j in range(8):
        pos = offv + (j * LANE) + iot
        pm = pos < kpad_v
        prow = lax.bitwise_and(lax.shift_right_logical(pos, 7), GMAX - 1)
        pcol = lax.bitwise_and(pos, c127)
        plsc.store_scatter(stg_src, [prow, pcol], zero16, mask=pm)
        plsc.store_scatter(stg_dst, [prow, pcol], trash, mask=pm)
      flush_range(fg, ngt)

      plsc.subcore_barrier()
      pltpu.sync_copy(accum.at[pl.ds(s * RPT, RPT)],
                      out_hbm.at[pl.ds(lo + s * RPT, RPT)])
      plsc.subcore_barrier()

  return kern


def _tc_layer1(s1cnt, x, wcat, b1):
  R = 1000
  grid = (N // R,)

  def body(s1_ref, x_ref, w_ref, b_ref, out_ref):
    s1 = s1_ref[...]
    cnt = jnp.maximum(s1[:, 4:5], 1.0)
    feat = jnp.concatenate([s1[:, 0:4] / cnt, x_ref[...]], axis=1)
    h = lax.dot_general(feat, w_ref[...], (((1,), (1,)), ((), ())),
                        preferred_element_type=jnp.float32)
    out_ref[...] = jnp.maximum(h + b_ref[...], 0.0)

  return pl.pallas_call(
      body,
      grid=grid,
      in_specs=[
          pl.BlockSpec((R, 8), lambda i: (i, 0)),
          pl.BlockSpec((R, 4), lambda i: (i, 0)),
          pl.BlockSpec((HID, 8), lambda i: (0, 0)),
          pl.BlockSpec((1, HID), lambda i: (0, 0)),
      ],
      out_specs=pl.BlockSpec((R, HID), lambda i: (i, 0)),
      out_shape=jax.ShapeDtypeStruct((N, HID), jnp.float32),
  )(s1cnt, x, wcat, b1)


def _tc_layer2_head(s2, h1, cnt, W2l, b2, W2r, Wp, bp, Wc1, bc1, Wc2, bc2):
  R = 1000
  nblk = N // R

  def body(s2_ref, h1_ref, cnt_ref, w2l_ref, w2r_ref, b2_ref,
           wp_ref, bp_ref, wc1_ref, bc1_ref, wc2_ref, bc2_ref,
           out_ref, acc_ref):
    i = pl.program_id(0)

    @pl.when(i == 0)
    def _():
      acc_ref[...] = jnp.zeros_like(acc_ref)

    cnt = jnp.maximum(cnt_ref[...], 1.0)
    mean = s2_ref[...] / cnt
    h = (lax.dot_general(mean, w2l_ref[...], (((1,), (1,)), ((), ())),
                         preferred_element_type=jnp.float32)
         + lax.dot_general(h1_ref[...], w2r_ref[...],
                           (((1,), (1,)), ((), ())),
                           preferred_element_type=jnp.float32)
         + b2_ref[...])
    h2 = jnp.maximum(h, 0.0)
    acc_ref[...] += jnp.sum(h2, axis=0, keepdims=True)

    @pl.when(i == nblk - 1)
    def _():
      pooled = acc_ref[...] / float(N)
      emb = jnp.maximum(
          lax.dot_general(pooled, wp_ref[...], (((1,), (1,)), ((), ())),
                          preferred_element_type=jnp.float32) + bp_ref[...],
          0.0)
      hc = jnp.maximum(
          lax.dot_general(emb, wc1_ref[...], (((1,), (1,)), ((), ())),
                          preferred_element_type=jnp.float32) + bc1_ref[...],
          0.0)
      logit = jnp.sum(hc * wc2_ref[...], axis=1, keepdims=True) + bc2_ref[...]
      out_ref[...] = jax.nn.sigmoid(logit)

  return pl.pallas_call(
      body,
      grid=(nblk,),
      in_specs=[
          pl.BlockSpec((R, HID), lambda i: (i, 0)),
          pl.BlockSpec((R, HID), lambda i: (i, 0)),
          pl.BlockSpec((R, 1), lambda i: (i, 0)),
          pl.BlockSpec((HID, HID), lambda i: (0, 0)),
          pl.BlockSpec((HID, HID), lambda i: (0, 0)),
          pl.BlockSpec((1, HID), lambda i: (0, 0)),
          pl.BlockSpec((256, HID), lambda i: (0, 0)),
          pl.BlockSpec((1, 256), lambda i: (0, 0)),
          pl.BlockSpec((HID, 256), lambda i: (0, 0)),
          pl.BlockSpec((1, HID), lambda i: (0, 0)),
          pl.BlockSpec((1, HID), lambda i: (0, 0)),
          pl.BlockSpec((1, 1), lambda i: (0, 0)),
      ],
      out_specs=pl.BlockSpec((1, 1), lambda i: (0, 0)),
      out_shape=jax.ShapeDtypeStruct((1, 1), jnp.float32),
      scratch_shapes=[pltpu.VMEM((1, HID), jnp.float32)],
  )(s2, h1, cnt, W2l, W2r, b2, Wp, bp, Wc1, bc1, Wc2, bc2)


def kernel(x, edge_index, W1l, b1, W1r, W2l, b2, W2r, Wp, bp, Wc1, bc1,
           Wc2, bc2):
  src = edge_index[0].astype(jnp.int32)
  dst = edge_index[1].astype(jnp.int32)
  x = x.astype(jnp.float32)

  xp = jnp.concatenate(
      [x, jnp.ones((N, 1), jnp.float32), jnp.zeros((N, 3), jnp.float32)],
      axis=1)

  NP1 = 25088
  NP2 = 12544
  B = 2000

  zeros8 = jnp.zeros((128, 8), jnp.float32)
  zeros128 = jnp.zeros((128, HID), jnp.float32)

  seg1 = _build_seg_sum(N, 8, NP1, 1, B)
  s1cnt = seg1(src, dst, xp, zeros8)[:N]

  wcat = jnp.concatenate([W1l, W1r], axis=1)
  h1 = _tc_layer1(s1cnt, x, wcat, b1.reshape(1, HID))

  seg2 = _build_seg_sum(N, HID, NP2, 2, B)
  s2 = seg2(src, dst, h1, zeros128)[:N]

  cnt = s1cnt[:, 4:5]
  prob = _tc_layer2_head(
      s2, h1, cnt, W2l, b2.reshape(1, HID), W2r, Wp, bp.reshape(1, 256),
      Wc1, bc1.reshape(1, HID), Wc2, bc2.reshape(1, 1))
  return prob


# speedup vs baseline: 10.7032x; 1.1062x over previous
import functools

import jax
import jax.numpy as jnp
from jax import lax
from jax.experimental import pallas as pl
from jax.experimental.pallas import tpu as pltpu
from jax.experimental.pallas import tpu_sc as plsc

N = 50000
E = 800000
HID = 128

NC = 2
NS = 16
LANE = 16


def _build_seg_sum(n_table_rows, W, NP, P, B):
  GMAX = 32
  EPT = E // NS
  NB = EPT // B
  NVR = B // LANE
  ACC_R = NP + 128
  ZCH = ACC_R // 128
  ZPT = -(-ZCH // NS)
  RPT = NP // NS
  assert B + 128 <= GMAX * 128

  mesh = plsc.VectorSubcoreMesh(core_axis_name="c", subcore_axis_name="s")

  @functools.partial(
      pl.kernel,
      out_type=jax.ShapeDtypeStruct((NC * P * NP, W), jnp.float32),
      mesh=mesh,
      scratch_types=[
          pltpu.VMEM((2 * B,), jnp.int32),
          pltpu.VMEM((2 * B,), jnp.int32),
          pltpu.VMEM((GMAX, 128), jnp.int32),
          pltpu.VMEM((GMAX, 128), jnp.int32),
          pltpu.VMEM((256, W), jnp.float32),
          pltpu.VMEM_SHARED((ACC_R, W), jnp.float32),
          pltpu.SemaphoreType.DMA,
          pltpu.SemaphoreType.DMA,
          pltpu.SemaphoreType.DMA,
      ],
      compiler_params=pltpu.CompilerParams(
          needs_layout_passes=False, use_tc_tiling_on_sc=False),
  )
  def kern(src_hbm, dst_hbm, table_hbm, zeros_hbm, out_hbm,
           eb_src, eb_dst, stg_src, stg_dst, rowbuf, accum,
           sem_a, sem_b, sem_e):
    c = lax.axis_index("c")
    s = lax.axis_index("s")
    tile_e0 = s * EPT
    c127 = jnp.full((LANE,), 127, jnp.int32)
    iot = lax.iota(jnp.int32, LANE)
    trash = jnp.full((LANE,), NP, jnp.int32)
    zero16 = jnp.zeros((LANE,), jnp.int32)

    def issue_gather(g):
      rr = lax.bitwise_and(g, GMAX - 1)
      par = lax.bitwise_and(g, 1)

      @pl.when(par == 0)
      def _():
        pltpu.async_copy(table_hbm.at[stg_src.at[rr]],
                         rowbuf.at[pl.ds(0, 128)], sem_a)

      @pl.when(par == 1)
      def _():
        pltpu.async_copy(table_hbm.at[stg_src.at[rr]],
                         rowbuf.at[pl.ds(128, 128)], sem_b)

    def flush_range(g0, g1):
      @pl.when(g1 > g0)
      def _():
        issue_gather(g0)

      def flush(g, cc):
        @pl.when(g + 1 < g1)
        def _():
          issue_gather(g + 1)

        rr = lax.bitwise_and(g, GMAX - 1)
        par = lax.bitwise_and(g, 1)

        @pl.when(par == 0)
        def _():
          pltpu.make_async_copy(table_hbm.at[stg_src.at[rr]],
                                rowbuf.at[pl.ds(0, 128)], sem_a).wait()
          pltpu.sync_copy(rowbuf.at[pl.ds(0, 128)],
                          accum.at[stg_dst.at[rr]], add=True)

        @pl.when(par == 1)
        def _():
          pltpu.make_async_copy(table_hbm.at[stg_src.at[rr]],
                                rowbuf.at[pl.ds(128, 128)], sem_b).wait()
          pltpu.sync_copy(rowbuf.at[pl.ds(128, 128)],
                          accum.at[stg_dst.at[rr]], add=True)

        return cc

      lax.fori_loop(g0, g1, flush, 0)

    def issue_edges(blk):
      boff = lax.bitwise_and(blk, 1) * B
      base = tile_e0 + blk * B
      pltpu.async_copy(src_hbm.at[pl.ds(base, B)],
                       eb_src.at[pl.ds(boff, B)], sem_e)
      pltpu.async_copy(dst_hbm.at[pl.ds(base, B)],
                       eb_dst.at[pl.ds(boff, B)], sem_e)

    for p in range(P):
      r = c * P + p
      lo = r * NP
      lo_v = jnp.full((LANE,), 1, jnp.int32) * lo
      hi_v = lo_v + NP

      pltpu.sync_copy(zeros_hbm, rowbuf.at[pl.ds(0, 128)])
      for j in range(ZPT):
        ch = s * ZPT + j

        @pl.when(ch < ZCH)
        def _():
          pltpu.sync_copy(rowbuf.at[pl.ds(0, 128)],
                          accum.at[pl.ds(ch * 128, 128)])

      plsc.subcore_barrier()
      issue_edges(0)

      def block_body(blk, carry):
        offv, fg = carry
        boff = lax.bitwise_and(blk, 1) * B
        base = tile_e0 + blk * B
        pltpu.make_async_copy(src_hbm.at[pl.ds(base, B)],
                              eb_src.at[pl.ds(boff, B)], sem_e).wait()
        pltpu.make_async_copy(dst_hbm.at[pl.ds(base, B)],
                              eb_dst.at[pl.ds(boff, B)], sem_e).wait()

        @pl.when(blk + 1 < NB)
        def _():
          issue_edges(blk + 1)

        def scan_body(i, off):
          d = eb_dst[pl.ds(boff + i * LANE, LANE)]
          sv = eb_src[pl.ds(boff + i * LANE, LANE)]
          m = (d >= lo_v) & (d < hi_v)
          inc = jnp.where(m, 1, 0).astype(jnp.int32)
          pos = off + plsc.cumsum(inc) - 1
          row = lax.bitwise_and(lax.shift_right_logical(pos, 7), GMAX - 1)
          col = lax.bitwise_and(pos, c127)
          plsc.store_scatter(stg_src, [row, col], sv, mask=m)
          plsc.store_scatter(stg_dst, [row, col], d - lo_v, mask=m)
          return off + plsc.all_reduce_population_count(m)

        offv2 = lax.fori_loop(0, NVR, scan_body, offv)
        tg = lax.shift_right_logical(jnp.max(offv2), 7)
        flush_range(fg, tg)
        return offv2, tg

      offv, fg = lax.fori_loop(
          0, NB, block_body,
          (jnp.zeros((LANE,), jnp.int32), jnp.int32(0)))

      k = jnp.max(offv)
      ngt = lax.shift_right_logical(k + 127, 7)
      kpad_v = jnp.zeros((LANE,), jnp.int32) + ngt * 128
      for j in range(8):
        pos = offv + (j * LANE) + iot
        pm = pos < kpad_v
        prow = lax.bitwise_and(lax.shift_right_logical(pos, 7), GMAX - 1)
        pcol = lax.bitwise_and(pos, c127)
        plsc.store_scatter(stg_src, [prow, pcol], zero16, mask=pm)
        plsc.store_scatter(stg_dst, [prow, pcol], trash, mask=pm)
      flush_range(fg, ngt)

      plsc.subcore_barrier()
      pltpu.sync_copy(accum.at[pl.ds(s * RPT, RPT)],
                      out_hbm.at[pl.ds(lo + s * RPT, RPT)])
      plsc.subcore_barrier()

  return kern


def _tc_layer1(s1cnt, x, wcat, b1):
  R = 1000
  grid = (N // R,)

  def body(s1_ref, x_ref, w_ref, b_ref, out_ref):
    s1 = s1_ref[...]
    cnt = jnp.maximum(s1[:, 4:5], 1.0)
    feat = jnp.concatenate([s1[:, 0:4] / cnt, x_ref[...]], axis=1)
    h = lax.dot_general(feat, w_ref[...], (((1,), (1,)), ((), ())),
                        preferred_element_type=jnp.float32)
    out_ref[...] = jnp.maximum(h + b_ref[...], 0.0)

  return pl.pallas_call(
      body,
      grid=grid,
      in_specs=[
          pl.BlockSpec((R, 8), lambda i: (i, 0)),
          pl.BlockSpec((R, 4), lambda i: (i, 0)),
          pl.BlockSpec((HID, 8), lambda i: (0, 0)),
          pl.BlockSpec((1, HID), lambda i: (0, 0)),
      ],
      out_specs=pl.BlockSpec((R, HID), lambda i: (i, 0)),
      out_shape=jax.ShapeDtypeStruct((N, HID), jnp.float32),
  )(s1cnt, x, wcat, b1)


def _tc_layer2_head(s2, h1, cnt, W2l, b2, W2r, Wp, bp, Wc1, bc1, Wc2, bc2):
  R = 1000
  nblk = N // R

  def body(s2_ref, h1_ref, cnt_ref, w2l_ref, w2r_ref, b2_ref,
           wp_ref, bp_ref, wc1_ref, bc1_ref, wc2_ref, bc2_ref,
           out_ref, acc_ref):
    i = pl.program_id(0)

    @pl.when(i == 0)
    def _():
      acc_ref[...] = jnp.zeros_like(acc_ref)

    cnt = jnp.maximum(cnt_ref[...], 1.0)
    mean = s2_ref[...] / cnt
    h = (lax.dot_general(mean, w2l_ref[...], (((1,), (1,)), ((), ())),
                         preferred_element_type=jnp.float32)
         + lax.dot_general(h1_ref[...], w2r_ref[...],
                           (((1,), (1,)), ((), ())),
                           preferred_element_type=jnp.float32)
         + b2_ref[...])
    h2 = jnp.maximum(h, 0.0)
    acc_ref[...] += jnp.sum(h2, axis=0, keepdims=True)

    @pl.when(i == nblk - 1)
    def _():
      pooled = acc_ref[...] / float(N)
      emb = jnp.maximum(
          lax.dot_general(pooled, wp_ref[...], (((1,), (1,)), ((), ())),
                          preferred_element_type=jnp.float32) + bp_ref[...],
          0.0)
      hc = jnp.maximum(
          lax.dot_general(emb, wc1_ref[...], (((1,), (1,)), ((), ())),
                          preferred_element_type=jnp.float32) + bc1_ref[...],
          0.0)
      logit = jnp.sum(hc * wc2_ref[...], axis=1, keepdims=True) + bc2_ref[...]
      out_ref[...] = jax.nn.sigmoid(logit)

  return pl.pallas_call(
      body,
      grid=(nblk,),
      in_specs=[
          pl.BlockSpec((R, HID), lambda i: (i, 0)),
          pl.BlockSpec((R, HID), lambda i: (i, 0)),
          pl.BlockSpec((R, 1), lambda i: (i, 0)),
          pl.BlockSpec((HID, HID), lambda i: (0, 0)),
          pl.BlockSpec((HID, HID), lambda i: (0, 0)),
          pl.BlockSpec((1, HID), lambda i: (0, 0)),
          pl.BlockSpec((256, HID), lambda i: (0, 0)),
          pl.BlockSpec((1, 256), lambda i: (0, 0)),
          pl.BlockSpec((HID, 256), lambda i: (0, 0)),
          pl.BlockSpec((1, HID), lambda i: (0, 0)),
          pl.BlockSpec((1, HID), lambda i: (0, 0)),
          pl.BlockSpec((1, 1), lambda i: (0, 0)),
      ],
      out_specs=pl.BlockSpec((1, 1), lambda i: (0, 0)),
      out_shape=jax.ShapeDtypeStruct((1, 1), jnp.float32),
      scratch_shapes=[pltpu.VMEM((1, HID), jnp.float32)],
  )(s2, h1, cnt, W2l, W2r, b2, Wp, bp, Wc1, bc1, Wc2, bc2)


def kernel(x, edge_index, W1l, b1, W1r, W2l, b2, W2r, Wp, bp, Wc1, bc1,
           Wc2, bc2):
  src = edge_index[0].astype(jnp.int32)
  dst = edge_index[1].astype(jnp.int32)
  x = x.astype(jnp.float32)

  xp = jnp.concatenate(
      [x, jnp.ones((N, 1), jnp.float32), jnp.zeros((N, 3), jnp.float32)],
      axis=1)

  NP1 = 25088
  NP2 = 8448
  B = 2000

  zeros8 = jnp.zeros((128, 8), jnp.float32)
  zeros128 = jnp.zeros((128, HID), jnp.float32)

  seg1 = _build_seg_sum(N, 8, NP1, 1, B)
  s1cnt = seg1(src, dst, xp, zeros8)[:N]

  wcat = jnp.concatenate([W1l, W1r], axis=1)
  h1 = _tc_layer1(s1cnt, x, wcat, b1.reshape(1, HID))

  seg2 = _build_seg_sum(N, HID, NP2, 3, B)
  s2 = seg2(src, dst, h1, zeros128)[:N]

  cnt = s1cnt[:, 4:5]
  prob = _tc_layer2_head(
      s2, h1, cnt, W2l, b2.reshape(1, HID), W2r, Wp, bp.reshape(1, 256),
      Wc1, bc1.reshape(1, HID), Wc2, bc2.reshape(1, 1))
  return prob


# speedup vs baseline: 11.2120x; 1.0475x over previous
import functools

import jax
import jax.numpy as jnp
from jax import lax
from jax.experimental import pallas as pl
from jax.experimental.pallas import tpu as pltpu
from jax.experimental.pallas import tpu_sc as plsc

N = 50000
E = 800000
HID = 128

NC = 2
NS = 16
LANE = 16


def _build_seg_sum(n_table_rows, W, NP, P, B):
  GMAX = 32
  EPT = E // NS
  NB = EPT // B
  NVR = B // LANE
  ACC_R = NP + 128
  ZCH = ACC_R // 128
  ZPT = -(-ZCH // NS)
  RPT = NP // NS
  assert B + 128 <= GMAX * 128

  mesh = plsc.VectorSubcoreMesh(core_axis_name="c", subcore_axis_name="s")

  @functools.partial(
      pl.kernel,
      out_type=jax.ShapeDtypeStruct((NC * P * NP, W), jnp.float32),
      mesh=mesh,
      scratch_types=[
          pltpu.VMEM((2 * B,), jnp.int32),
          pltpu.VMEM((2 * B,), jnp.int32),
          pltpu.VMEM((GMAX, 128), jnp.int32),
          pltpu.VMEM((GMAX, 128), jnp.int32),
          pltpu.VMEM((256, W), jnp.float32),
          pltpu.VMEM_SHARED((ACC_R, W), jnp.float32),
          pltpu.SemaphoreType.DMA,
          pltpu.SemaphoreType.DMA,
          pltpu.SemaphoreType.DMA,
      ],
      compiler_params=pltpu.CompilerParams(
          needs_layout_passes=False, use_tc_tiling_on_sc=False),
  )
  def kern(src_hbm, dst_hbm, table_hbm, zeros_hbm, out_hbm,
           eb_src, eb_dst, stg_src, stg_dst, rowbuf, accum,
           sem_a, sem_b, sem_e):
    c = lax.axis_index("c")
    s = lax.axis_index("s")
    tile_e0 = s * EPT
    c127 = jnp.full((LANE,), 127, jnp.int32)
    iot = lax.iota(jnp.int32, LANE)
    trash = jnp.full((LANE,), NP, jnp.int32)
    zero16 = jnp.zeros((LANE,), jnp.int32)

    def issue_gather(g):
      rr = lax.bitwise_and(g, GMAX - 1)
      par = lax.bitwise_and(g, 1)

      @pl.when(par == 0)
      def _():
        pltpu.async_copy(table_hbm.at[stg_src.at[rr]],
                         rowbuf.at[pl.ds(0, 128)], sem_a)

      @pl.when(par == 1)
      def _():
        pltpu.async_copy(table_hbm.at[stg_src.at[rr]],
                         rowbuf.at[pl.ds(128, 128)], sem_b)

    def drain_one(g):
      rr = lax.bitwise_and(g, GMAX - 1)
      par = lax.bitwise_and(g, 1)

      @pl.when(par == 0)
      def _():
        pltpu.make_async_copy(table_hbm.at[stg_src.at[rr]],
                              rowbuf.at[pl.ds(0, 128)], sem_a).wait()
        pltpu.sync_copy(rowbuf.at[pl.ds(0, 128)],
                        accum.at[stg_dst.at[rr]], add=True)

      @pl.when(par == 1)
      def _():
        pltpu.make_async_copy(table_hbm.at[stg_src.at[rr]],
                              rowbuf.at[pl.ds(128, 128)], sem_b).wait()
        pltpu.sync_copy(rowbuf.at[pl.ds(128, 128)],
                        accum.at[stg_dst.at[rr]], add=True)

    def drain_range(g0, g1):
      def body(g, cc):
        drain_one(g)
        return cc

      lax.fori_loop(g0, g1, body, 0)

    def issue_edges(blk):
      boff = lax.bitwise_and(blk, 1) * B
      base = tile_e0 + blk * B
      pltpu.async_copy(src_hbm.at[pl.ds(base, B)],
                       eb_src.at[pl.ds(boff, B)], sem_e)
      pltpu.async_copy(dst_hbm.at[pl.ds(base, B)],
                       eb_dst.at[pl.ds(boff, B)], sem_e)

    for p in range(P):
      r = c * P + p
      lo = r * NP
      lo_v = jnp.full((LANE,), 1, jnp.int32) * lo
      hi_v = lo_v + NP

      pltpu.sync_copy(zeros_hbm, rowbuf.at[pl.ds(0, 128)])
      for j in range(ZPT):
        ch = s * ZPT + j

        @pl.when(ch < ZCH)
        def _():
          pltpu.sync_copy(rowbuf.at[pl.ds(0, 128)],
                          accum.at[pl.ds(ch * 128, 128)])

      plsc.subcore_barrier()
      issue_edges(0)

      def block_body(blk, carry):
        offv, tg, gd = carry
        boff = lax.bitwise_and(blk, 1) * B
        base = tile_e0 + blk * B
        pltpu.make_async_copy(src_hbm.at[pl.ds(base, B)],
                              eb_src.at[pl.ds(boff, B)], sem_e).wait()
        pltpu.make_async_copy(dst_hbm.at[pl.ds(base, B)],
                              eb_dst.at[pl.ds(boff, B)], sem_e).wait()

        @pl.when(blk + 1 < NB)
        def _():
          issue_edges(blk + 1)

        def scan_body(i, carry_s):
          off, tgs, gds = carry_s
          d = eb_dst[pl.ds(boff + i * LANE, LANE)]
          sv = eb_src[pl.ds(boff + i * LANE, LANE)]
          m = (d >= lo_v) & (d < hi_v)
          inc = jnp.where(m, 1, 0).astype(jnp.int32)
          pos = off + plsc.cumsum(inc) - 1
          row = lax.bitwise_and(lax.shift_right_logical(pos, 7), GMAX - 1)
          col = lax.bitwise_and(pos, c127)
          plsc.store_scatter(stg_src, [row, col], sv, mask=m)
          plsc.store_scatter(stg_dst, [row, col], d - lo_v, mask=m)
          off2 = off + plsc.all_reduce_population_count(m)
          cond = jnp.any(off2 >= (tgs + 1) * 128)
          full = jnp.logical_and(cond, (tgs - gds) >= 2)

          @pl.when(full)
          def _():
            drain_one(gds)

          @pl.when(cond)
          def _():
            issue_gather(tgs)

          gds2 = jnp.where(full, gds + 1, gds)
          tgs2 = jnp.where(cond, tgs + 1, tgs)
          return off2, tgs2, gds2

        return lax.fori_loop(0, NVR, scan_body, (offv, tg, gd))

      offv, tg, gd = lax.fori_loop(
          0, NB, block_body,
          (jnp.zeros((LANE,), jnp.int32), jnp.int32(0), jnp.int32(0)))

      k = jnp.max(offv)
      ngt = lax.shift_right_logical(k + 127, 7)
      kpad_v = jnp.zeros((LANE,), jnp.int32) + ngt * 128
      for j in range(8):
        pos = offv + (j * LANE) + iot
        pm = pos < kpad_v
        prow = lax.bitwise_and(lax.shift_right_logical(pos, 7), GMAX - 1)
        pcol = lax.bitwise_and(pos, c127)
        plsc.store_scatter(stg_src, [prow, pcol], zero16, mask=pm)
        plsc.store_scatter(stg_dst, [prow, pcol], trash, mask=pm)

      @pl.when(ngt > tg)
      def _():
        issue_gather(tg)

      drain_range(gd, ngt)

      plsc.subcore_barrier()
      pltpu.sync_copy(accum.at[pl.ds(s * RPT, RPT)],
                      out_hbm.at[pl.ds(lo + s * RPT, RPT)])
      plsc.subcore_barrier()

  return kern


def _tc_layer1(s1cnt, x, wcat, b1):
  R = 1000
  grid = (N // R,)

  def body(s1_ref, x_ref, w_ref, b_ref, out_ref):
    s1 = s1_ref[...]
    cnt = jnp.maximum(s1[:, 4:5], 1.0)
    feat = jnp.concatenate([s1[:, 0:4] / cnt, x_ref[...]], axis=1)
    h = lax.dot_general(feat, w_ref[...], (((1,), (1,)), ((), ())),
                        preferred_element_type=jnp.float32)
    out_ref[...] = jnp.maximum(h + b_ref[...], 0.0)

  return pl.pallas_call(
      body,
      grid=grid,
      in_specs=[
          pl.BlockSpec((R, 8), lambda i: (i, 0)),
          pl.BlockSpec((R, 4), lambda i: (i, 0)),
          pl.BlockSpec((HID, 8), lambda i: (0, 0)),
          pl.BlockSpec((1, HID), lambda i: (0, 0)),
      ],
      out_specs=pl.BlockSpec((R, HID), lambda i: (i, 0)),
      out_shape=jax.ShapeDtypeStruct((N, HID), jnp.float32),
  )(s1cnt, x, wcat, b1)


def _tc_layer2_head(s2, h1, cnt, W2l, b2, W2r, Wp, bp, Wc1, bc1, Wc2, bc2):
  R = 1000
  nblk = N // R

  def body(s2_ref, h1_ref, cnt_ref, w2l_ref, w2r_ref, b2_ref,
           wp_ref, bp_ref, wc1_ref, bc1_ref, wc2_ref, bc2_ref,
           out_ref, acc_ref):
    i = pl.program_id(0)

    @pl.when(i == 0)
    def _():
      acc_ref[...] = jnp.zeros_like(acc_ref)

    cnt = jnp.maximum(cnt_ref[...], 1.0)
    mean = s2_ref[...] / cnt
    h = (lax.dot_general(mean, w2l_ref[...], (((1,), (1,)), ((), ())),
                         preferred_element_type=jnp.float32)
         + lax.dot_general(h1_ref[...], w2r_ref[...],
                           (((1,), (1,)), ((), ())),
                           preferred_element_type=jnp.float32)
         + b2_ref[...])
    h2 = jnp.maximum(h, 0.0)
    acc_ref[...] += jnp.sum(h2, axis=0, keepdims=True)

    @pl.when(i == nblk - 1)
    def _():
      pooled = acc_ref[...] / float(N)
      emb = jnp.maximum(
          lax.dot_general(pooled, wp_ref[...], (((1,), (1,)), ((), ())),
                          preferred_element_type=jnp.float32) + bp_ref[...],
          0.0)
      hc = jnp.maximum(
          lax.dot_general(emb, wc1_ref[...], (((1,), (1,)), ((), ())),
                          preferred_element_type=jnp.float32) + bc1_ref[...],
          0.0)
      logit = jnp.sum(hc * wc2_ref[...], axis=1, keepdims=True) + bc2_ref[...]
      out_ref[...] = jax.nn.sigmoid(logit)

  return pl.pallas_call(
      body,
      grid=(nblk,),
      in_specs=[
          pl.BlockSpec((R, HID), lambda i: (i, 0)),
          pl.BlockSpec((R, HID), lambda i: (i, 0)),
          pl.BlockSpec((R, 1), lambda i: (i, 0)),
          pl.BlockSpec((HID, HID), lambda i: (0, 0)),
          pl.BlockSpec((HID, HID), lambda i: (0, 0)),
          pl.BlockSpec((1, HID), lambda i: (0, 0)),
          pl.BlockSpec((256, HID), lambda i: (0, 0)),
          pl.BlockSpec((1, 256), lambda i: (0, 0)),
          pl.BlockSpec((HID, 256), lambda i: (0, 0)),
          pl.BlockSpec((1, HID), lambda i: (0, 0)),
          pl.BlockSpec((1, HID), lambda i: (0, 0)),
          pl.BlockSpec((1, 1), lambda i: (0, 0)),
      ],
      out_specs=pl.BlockSpec((1, 1), lambda i: (0, 0)),
      out_shape=jax.ShapeDtypeStruct((1, 1), jnp.float32),
      scratch_shapes=[pltpu.VMEM((1, HID), jnp.float32)],
  )(s2, h1, cnt, W2l, W2r, b2, Wp, bp, Wc1, bc1, Wc2, bc2)


def kernel(x, edge_index, W1l, b1, W1r, W2l, b2, W2r, Wp, bp, Wc1, bc1,
           Wc2, bc2):
  src = edge_index[0].astype(jnp.int32)
  dst = edge_index[1].astype(jnp.int32)
  x = x.astype(jnp.float32)

  xp = jnp.concatenate(
      [x, jnp.ones((N, 1), jnp.float32), jnp.zeros((N, 3), jnp.float32)],
      axis=1)

  NP1 = 25088
  NP2 = 8448
  B = 2000

  zeros8 = jnp.zeros((128, 8), jnp.float32)
  zeros128 = jnp.zeros((128, HID), jnp.float32)

  seg1 = _build_seg_sum(N, 8, NP1, 1, B)
  s1cnt = seg1(src, dst, xp, zeros8)[:N]

  wcat = jnp.concatenate([W1l, W1r], axis=1)
  h1 = _tc_layer1(s1cnt, x, wcat, b1.reshape(1, HID))

  seg2 = _build_seg_sum(N, HID, NP2, 3, B)
  s2 = seg2(src, dst, h1, zeros128)[:N]

  cnt = s1cnt[:, 4:5]
  prob = _tc_layer2_head(
      s2, h1, cnt, W2l, b2.reshape(1, HID), W2r, Wp, bp.reshape(1, 256),
      Wc1, bc1.reshape(1, HID), Wc2, bc2.reshape(1, 1))
  return prob


# speedup vs baseline: 14.4486x; 1.2887x over previous
import functools

import jax
import jax.numpy as jnp
from jax import lax
from jax.experimental import pallas as pl
from jax.experimental.pallas import tpu as pltpu
from jax.experimental.pallas import tpu_sc as plsc

N = 50000
E = 800000
HID = 128

NC = 2
NS = 16
LANE = 16


def _build_seg_sum(n_table_rows, W, NP, P, B, dt):
  GMAX = 32
  EPT = E // NS
  NB = EPT // B
  NVR = B // LANE
  ACC_R = NP + 128
  ZCH = ACC_R // 128
  ZPT = -(-ZCH // NS)
  RPT = NP // NS
  assert B + 128 <= GMAX * 128

  mesh = plsc.VectorSubcoreMesh(core_axis_name="c", subcore_axis_name="s")

  @functools.partial(
      pl.kernel,
      out_type=jax.ShapeDtypeStruct((NC * P * NP, W), dt),
      mesh=mesh,
      scratch_types=[
          pltpu.VMEM((2 * B,), jnp.int32),
          pltpu.VMEM((2 * B,), jnp.int32),
          pltpu.VMEM((GMAX, 128), jnp.int32),
          pltpu.VMEM((GMAX, 128), jnp.int32),
          pltpu.VMEM((256, W), dt),
          pltpu.VMEM_SHARED((ACC_R, W), dt),
          pltpu.SemaphoreType.DMA,
          pltpu.SemaphoreType.DMA,
          pltpu.SemaphoreType.DMA,
      ],
      compiler_params=pltpu.CompilerParams(
          needs_layout_passes=False, use_tc_tiling_on_sc=False),
  )
  def kern(src_hbm, dst_hbm, table_hbm, zeros_hbm, out_hbm,
           eb_src, eb_dst, stg_src, stg_dst, rowbuf, accum,
           sem_a, sem_b, sem_e):
    c = lax.axis_index("c")
    s = lax.axis_index("s")
    tile_e0 = s * EPT
    c127 = jnp.full((LANE,), 127, jnp.int32)
    iot = lax.iota(jnp.int32, LANE)
    trash = jnp.full((LANE,), NP, jnp.int32)
    zero16 = jnp.zeros((LANE,), jnp.int32)

    def issue_gather(g):
      rr = lax.bitwise_and(g, GMAX - 1)
      par = lax.bitwise_and(g, 1)

      @pl.when(par == 0)
      def _():
        pltpu.async_copy(table_hbm.at[stg_src.at[rr]],
                         rowbuf.at[pl.ds(0, 128)], sem_a)

      @pl.when(par == 1)
      def _():
        pltpu.async_copy(table_hbm.at[stg_src.at[rr]],
                         rowbuf.at[pl.ds(128, 128)], sem_b)

    def drain_one(g):
      rr = lax.bitwise_and(g, GMAX - 1)
      par = lax.bitwise_and(g, 1)

      @pl.when(par == 0)
      def _():
        pltpu.make_async_copy(table_hbm.at[stg_src.at[rr]],
                              rowbuf.at[pl.ds(0, 128)], sem_a).wait()
        pltpu.sync_copy(rowbuf.at[pl.ds(0, 128)],
                        accum.at[stg_dst.at[rr]], add=True)

      @pl.when(par == 1)
      def _():
        pltpu.make_async_copy(table_hbm.at[stg_src.at[rr]],
                              rowbuf.at[pl.ds(128, 128)], sem_b).wait()
        pltpu.sync_copy(rowbuf.at[pl.ds(128, 128)],
                        accum.at[stg_dst.at[rr]], add=True)

    def drain_range(g0, g1):
      def body(g, cc):
        drain_one(g)
        return cc

      lax.fori_loop(g0, g1, body, 0)

    def issue_edges(blk):
      boff = lax.bitwise_and(blk, 1) * B
      base = tile_e0 + blk * B
      pltpu.async_copy(src_hbm.at[pl.ds(base, B)],
                       eb_src.at[pl.ds(boff, B)], sem_e)
      pltpu.async_copy(dst_hbm.at[pl.ds(base, B)],
                       eb_dst.at[pl.ds(boff, B)], sem_e)

    for p in range(P):
      r = c * P + p
      lo = r * NP
      lo_v = jnp.full((LANE,), 1, jnp.int32) * lo
      hi_v = lo_v + NP

      pltpu.sync_copy(zeros_hbm, rowbuf.at[pl.ds(0, 128)])
      for j in range(ZPT):
        ch = s * ZPT + j

        @pl.when(ch < ZCH)
        def _():
          pltpu.sync_copy(rowbuf.at[pl.ds(0, 128)],
                          accum.at[pl.ds(ch * 128, 128)])

      plsc.subcore_barrier()
      issue_edges(0)

      def block_body(blk, carry):
        offv, tg, gd = carry
        boff = lax.bitwise_and(blk, 1) * B
        base = tile_e0 + blk * B
        pltpu.make_async_copy(src_hbm.at[pl.ds(base, B)],
                              eb_src.at[pl.ds(boff, B)], sem_e).wait()
        pltpu.make_async_copy(dst_hbm.at[pl.ds(base, B)],
                              eb_dst.at[pl.ds(boff, B)], sem_e).wait()

        @pl.when(blk + 1 < NB)
        def _():
          issue_edges(blk + 1)

        def scan_body(i, carry_s):
          off, tgs, gds = carry_s
          d = eb_dst[pl.ds(boff + i * LANE, LANE)]
          sv = eb_src[pl.ds(boff + i * LANE, LANE)]
          m = (d >= lo_v) & (d < hi_v)
          inc = jnp.where(m, 1, 0).astype(jnp.int32)
          pos = off + plsc.cumsum(inc) - 1
          row = lax.bitwise_and(lax.shift_right_logical(pos, 7), GMAX - 1)
          col = lax.bitwise_and(pos, c127)
          plsc.store_scatter(stg_src, [row, col], sv, mask=m)
          plsc.store_scatter(stg_dst, [row, col], d - lo_v, mask=m)
          off2 = off + plsc.all_reduce_population_count(m)
          cond = jnp.any(off2 >= (tgs + 1) * 128)
          full = jnp.logical_and(cond, (tgs - gds) >= 2)

          @pl.when(full)
          def _():
            drain_one(gds)

          @pl.when(cond)
          def _():
            issue_gather(tgs)

          gds2 = jnp.where(full, gds + 1, gds)
          tgs2 = jnp.where(cond, tgs + 1, tgs)
          return off2, tgs2, gds2

        return lax.fori_loop(0, NVR, scan_body, (offv, tg, gd))

      offv, tg, gd = lax.fori_loop(
          0, NB, block_body,
          (jnp.zeros((LANE,), jnp.int32), jnp.int32(0), jnp.int32(0)))

      k = jnp.max(offv)
      ngt = lax.shift_right_logical(k + 127, 7)
      kpad_v = jnp.zeros((LANE,), jnp.int32) + ngt * 128
      for j in range(8):
        pos = offv + (j * LANE) + iot
        pm = pos < kpad_v
        prow = lax.bitwise_and(lax.shift_right_logical(pos, 7), GMAX - 1)
        pcol = lax.bitwise_and(pos, c127)
        plsc.store_scatter(stg_src, [prow, pcol], zero16, mask=pm)
        plsc.store_scatter(stg_dst, [prow, pcol], trash, mask=pm)

      @pl.when(ngt > tg)
      def _():
        issue_gather(tg)

      drain_range(gd, ngt)

      plsc.subcore_barrier()
      pltpu.sync_copy(accum.at[pl.ds(s * RPT, RPT)],
                      out_hbm.at[pl.ds(lo + s * RPT, RPT)])
      plsc.subcore_barrier()

  return kern


def _tc_layer1(s1cnt, x, wcat, b1, npad):
  R = 1056
  grid = (npad // R,)

  def body(s1_ref, x_ref, w_ref, b_ref, out_ref):
    s1 = s1_ref[...]
    cnt = jnp.maximum(s1[:, 4:5], 1.0)
    feat = jnp.concatenate([s1[:, 0:4] / cnt, x_ref[...]], axis=1)
    h = lax.dot_general(feat, w_ref[...], (((1,), (1,)), ((), ())),
                        preferred_element_type=jnp.float32)
    out_ref[...] = jnp.maximum(h + b_ref[...], 0.0).astype(jnp.bfloat16)

  return pl.pallas_call(
      body,
      grid=grid,
      in_specs=[
          pl.BlockSpec((R, 8), lambda i: (i, 0)),
          pl.BlockSpec((R, 4), lambda i: (i, 0)),
          pl.BlockSpec((HID, 8), lambda i: (0, 0)),
          pl.BlockSpec((1, HID), lambda i: (0, 0)),
      ],
      out_specs=pl.BlockSpec((R, HID), lambda i: (i, 0)),
      out_shape=jax.ShapeDtypeStruct((npad, HID), jnp.bfloat16),
  )(s1cnt, x, wcat, b1)


def _tc_layer2_head(s2, h1, cnt, W2l, b2, W2r, Wp, bp, Wc1, bc1, Wc2, bc2,
                    npad):
  R = 1056
  nblk = npad // R

  def body(s2_ref, h1_ref, cnt_ref, w2l_ref, w2r_ref, b2_ref,
           wp_ref, bp_ref, wc1_ref, bc1_ref, wc2_ref, bc2_ref,
           out_ref, acc_ref):
    i = pl.program_id(0)

    @pl.when(i == 0)
    def _():
      acc_ref[...] = jnp.zeros_like(acc_ref)

    cnt = jnp.maximum(cnt_ref[...], 1.0)
    mean = s2_ref[...].astype(jnp.float32) / cnt
    h1b = h1_ref[...].astype(jnp.float32)
    h = (lax.dot_general(mean, w2l_ref[...], (((1,), (1,)), ((), ())),
                         preferred_element_type=jnp.float32)
         + lax.dot_general(h1b, w2r_ref[...],
                           (((1,), (1,)), ((), ())),
                           preferred_element_type=jnp.float32)
         + b2_ref[...])
    h2 = jnp.maximum(h, 0.0)
    rows = lax.broadcasted_iota(jnp.int32, (R, 1), 0) + i * R
    h2 = jnp.where(rows < N, h2, 0.0)
    acc_ref[...] += jnp.sum(h2, axis=0, keepdims=True)

    @pl.when(i == nblk - 1)
    def _():
      pooled = acc_ref[...] / float(N)
      emb = jnp.maximum(
          lax.dot_general(pooled, wp_ref[...], (((1,), (1,)), ((), ())),
                          preferred_element_type=jnp.float32) + bp_ref[...],
          0.0)
      hc = jnp.maximum(
          lax.dot_general(emb, wc1_ref[...], (((1,), (1,)), ((), ())),
                          preferred_element_type=jnp.float32) + bc1_ref[...],
          0.0)
      logit = jnp.sum(hc * wc2_ref[...], axis=1, keepdims=True) + bc2_ref[...]
      out_ref[...] = jax.nn.sigmoid(logit)

  return pl.pallas_call(
      body,
      grid=(nblk,),
      in_specs=[
          pl.BlockSpec((R, HID), lambda i: (i, 0)),
          pl.BlockSpec((R, HID), lambda i: (i, 0)),
          pl.BlockSpec((R, 1), lambda i: (i, 0)),
          pl.BlockSpec((HID, HID), lambda i: (0, 0)),
          pl.BlockSpec((HID, HID), lambda i: (0, 0)),
          pl.BlockSpec((1, HID), lambda i: (0, 0)),
          pl.BlockSpec((256, HID), lambda i: (0, 0)),
          pl.BlockSpec((1, 256), lambda i: (0, 0)),
          pl.BlockSpec((HID, 256), lambda i: (0, 0)),
          pl.BlockSpec((1, HID), lambda i: (0, 0)),
          pl.BlockSpec((1, HID), lambda i: (0, 0)),
          pl.BlockSpec((1, 1), lambda i: (0, 0)),
      ],
      out_specs=pl.BlockSpec((1, 1), lambda i: (0, 0)),
      out_shape=jax.ShapeDtypeStruct((1, 1), jnp.float32),
      scratch_shapes=[pltpu.VMEM((1, HID), jnp.float32)],
  )(s2, h1, cnt, W2l, W2r, b2, Wp, bp, Wc1, bc1, Wc2, bc2)


def kernel(x, edge_index, W1l, b1, W1r, W2l, b2, W2r, Wp, bp, Wc1, bc1,
           Wc2, bc2):
  src = edge_index[0].astype(jnp.int32)
  dst = edge_index[1].astype(jnp.int32)
  x = x.astype(jnp.float32)

  NP1 = 25344
  NP2 = 12672
  NPAD = 50688
  B = 2000

  xp = jnp.concatenate(
      [x, jnp.ones((N, 1), jnp.float32), jnp.zeros((N, 3), jnp.float32)],
      axis=1)
  xpad = jnp.pad(x, ((0, NPAD - N), (0, 0)))

  zeros8 = jnp.zeros((128, 8), jnp.float32)
  zeros128 = jnp.zeros((128, HID), jnp.bfloat16)

  seg1 = _build_seg_sum(N, 8, NP1, 1, B, jnp.float32)
  s1cnt = seg1(src, dst, xp, zeros8)

  wcat = jnp.concatenate([W1l, W1r], axis=1)
  h1 = _tc_layer1(s1cnt, xpad, wcat, b1.reshape(1, HID), NPAD)

  seg2 = _build_seg_sum(N, HID, NP2, 2, B, jnp.bfloat16)
  s2 = seg2(src, dst, h1, zeros128)

  cnt = s1cnt[:, 4:5]
  prob = _tc_layer2_head(
      s2, h1, cnt, W2l, b2.reshape(1, HID), W2r, Wp, bp.reshape(1, 256),
      Wc1, bc1.reshape(1, HID), Wc2, bc2.reshape(1, 1), NPAD)
  return prob
